# baseline (speedup 1.0000x reference)
import functools

import jax
import jax.numpy as jnp
from jax import lax
from jax.experimental import pallas as pl
from jax.experimental.pallas import tpu as pltpu
from jax.experimental.pallas import tpu_sc as plsc

_NUM_NODES = 48
_TOTAL = _NUM_NODES * _NUM_NODES
_LANES = 16
_TABLE_PAD = 16

_info = plsc.get_sparse_core_info()
_NC, _NS = _info.num_cores, _info.num_subcores
_NW = _NC * _NS
_PER_W = -(-_TOTAL // (_NW * _LANES)) * _LANES
_PADDED = _PER_W * _NW

_mesh = plsc.VectorSubcoreMesh(core_axis_name="c", subcore_axis_name="s")


@functools.partial(
    pl.kernel,
    mesh=_mesh,
    out_type=jax.ShapeDtypeStruct((_PADDED,), jnp.float32),
    scratch_types=[
        pltpu.VMEM((_TABLE_PAD,), jnp.float32),
        pltpu.VMEM((_PER_W,), jnp.int32),
        pltpu.VMEM((_PER_W,), jnp.float32),
    ],
    compiler_params=pltpu.CompilerParams(needs_layout_passes=False),
)
def _sc_embed_lookup(emb_hbm, idx_hbm, out_hbm, emb_v, idx_v, out_v):
    wid = lax.axis_index("s") * _NC + lax.axis_index("c")
    base = wid * _PER_W
    pltpu.sync_copy(emb_hbm, emb_v)
    pltpu.sync_copy(idx_hbm.at[pl.ds(base, _PER_W)], idx_v)
    for i in range(_PER_W // _LANES):
        iv = idx_v[pl.ds(i * _LANES, _LANES)]
        out_v[pl.ds(i * _LANES, _LANES)] = plsc.load_gather(emb_v, [iv])
    pltpu.sync_copy(out_v, out_hbm.at[pl.ds(base, _PER_W)])


@jax.jit
def kernel(dist_embed, dist_matrix):
    emb = jnp.zeros((_TABLE_PAD,), jnp.float32).at[: dist_embed.shape[0]].set(dist_embed)
    idx = jnp.pad(dist_matrix.reshape(-1).astype(jnp.int32), (0, _PADDED - _TOTAL))
    out = _sc_embed_lookup(emb, idx)
    return out[:_TOTAL].reshape(_NUM_NODES, _NUM_NODES)


# speedup vs baseline: 1.0914x; 1.0914x over previous
import functools

import jax
import jax.numpy as jnp
from jax import lax
from jax.experimental import pallas as pl
from jax.experimental.pallas import tpu as pltpu
from jax.experimental.pallas import tpu_sc as plsc

_N = 48
_TABLE = 14
_LANES = 16
_ROWS_PER_W = 2
_NWORK = _N // _ROWS_PER_W

_info = plsc.get_sparse_core_info()
_NC = _info.num_cores

_mesh = plsc.VectorSubcoreMesh(core_axis_name="c", subcore_axis_name="s")


@functools.partial(
    pl.kernel,
    mesh=_mesh,
    out_type=jax.ShapeDtypeStruct((_N, _N), jnp.float32),
    scratch_types=[
        pltpu.VMEM((_TABLE,), jnp.float32),
        pltpu.VMEM((_ROWS_PER_W, _N), jnp.int32),
        pltpu.VMEM((_ROWS_PER_W, _N), jnp.float32),
    ],
    compiler_params=pltpu.CompilerParams(needs_layout_passes=False),
)
def _sc_embed_lookup(emb_hbm, idx_hbm, out_hbm, emb_v, idx_v, out_v):
    wid = lax.axis_index("s") * _NC + lax.axis_index("c")

    @pl.when(wid < _NWORK)
    def _():
        base = wid * _ROWS_PER_W
        pltpu.sync_copy(emb_hbm, emb_v)
        pltpu.sync_copy(idx_hbm.at[pl.ds(base, _ROWS_PER_W), :], idx_v)
        for r in range(_ROWS_PER_W):
            for j in range(_N // _LANES):
                iv = idx_v[r, pl.ds(j * _LANES, _LANES)]
                out_v[r, pl.ds(j * _LANES, _LANES)] = plsc.load_gather(
                    emb_v, [iv]
                )
        pltpu.sync_copy(out_v, out_hbm.at[pl.ds(base, _ROWS_PER_W), :])


@jax.jit
def kernel(dist_embed, dist_matrix):
    return _sc_embed_lookup(dist_embed, dist_matrix.astype(jnp.int32))


# speedup vs baseline: 1.0996x; 1.0074x over previous
import functools

import jax
import jax.numpy as jnp
from jax import lax
from jax.experimental import pallas as pl
from jax.experimental.pallas import tpu as pltpu
from jax.experimental.pallas import tpu_sc as plsc

_N = 48
_TOTAL = _N * _N
_TABLE = 14
_LANES = 16
_NSUB = 16
_PER_W = _TOTAL // _NSUB

_mesh = plsc.VectorSubcoreMesh(
    core_axis_name="c", subcore_axis_name="s", num_cores=1
)


@functools.partial(
    pl.kernel,
    mesh=_mesh,
    out_type=jax.ShapeDtypeStruct((_TOTAL,), jnp.float32),
    scratch_types=[
        pltpu.VMEM((_TABLE,), jnp.float32),
        pltpu.VMEM((_PER_W,), jnp.int32),
        pltpu.VMEM((_PER_W,), jnp.float32),
        pltpu.SemaphoreType.DMA,
        pltpu.SemaphoreType.DMA,
    ],
    compiler_params=pltpu.CompilerParams(needs_layout_passes=False),
)
def _sc_embed_lookup(emb_hbm, idx_hbm, out_hbm, emb_v, idx_v, out_v, sem_e, sem_i):
    wid = lax.axis_index("s")
    base = wid * _PER_W
    cp_e = pltpu.make_async_copy(emb_hbm, emb_v, sem_e)
    cp_i = pltpu.make_async_copy(idx_hbm.at[pl.ds(base, _PER_W)], idx_v, sem_i)
    cp_e.start()
    cp_i.start()
    cp_e.wait()
    cp_i.wait()
    for j in range(_PER_W // _LANES):
        iv = idx_v[pl.ds(j * _LANES, _LANES)]
        out_v[pl.ds(j * _LANES, _LANES)] = plsc.load_gather(emb_v, [iv])
    pltpu.sync_copy(out_v, out_hbm.at[pl.ds(base, _PER_W)])


@jax.jit
def kernel(dist_embed, dist_matrix):
    flat = _sc_embed_lookup(dist_embed, dist_matrix.astype(jnp.int32).reshape(-1))
    return flat.reshape(_N, _N)


# speedup vs baseline: 1.1616x; 1.0564x over previous
import functools

import jax
import jax.numpy as jnp
from jax import lax
from jax.experimental import pallas as pl
from jax.experimental.pallas import tpu as pltpu
from jax.experimental.pallas import tpu_sc as plsc

_N = 48
_TABLE = 14
_LANES = 16
_ROWS_PER_W = 8
_NWORK = _N // _ROWS_PER_W

_mesh = plsc.VectorSubcoreMesh(
    core_axis_name="c", subcore_axis_name="s", num_cores=1
)


@functools.partial(
    pl.kernel,
    mesh=_mesh,
    out_type=jax.ShapeDtypeStruct((_N, _N), jnp.float32),
    scratch_types=[
        pltpu.VMEM((_TABLE,), jnp.float32),
        pltpu.VMEM((_ROWS_PER_W, _N), jnp.int32),
        pltpu.VMEM((_ROWS_PER_W, _N), jnp.float32),
    ],
    compiler_params=pltpu.CompilerParams(needs_layout_passes=False),
)
def _sc_embed_lookup(emb_hbm, idx_hbm, out_hbm, emb_v, idx_v, out_v):
    wid = lax.axis_index("s")

    @pl.when(wid < _NWORK)
    def _():
        base = wid * _ROWS_PER_W
        pltpu.sync_copy(emb_hbm, emb_v)
        pltpu.sync_copy(idx_hbm.at[pl.ds(base, _ROWS_PER_W), :], idx_v)

        def row(r, _):
            for j in range(_N // _LANES):
                iv = idx_v[r, pl.ds(j * _LANES, _LANES)]
                out_v[r, pl.ds(j * _LANES, _LANES)] = plsc.load_gather(
                    emb_v, [iv]
                )
            return _

        lax.fori_loop(0, _ROWS_PER_W, row, None)
        pltpu.sync_copy(out_v, out_hbm.at[pl.ds(base, _ROWS_PER_W), :])


@jax.jit
def kernel(dist_embed, dist_matrix):
    return _sc_embed_lookup(dist_embed, dist_matrix.astype(jnp.int32))
